# Initial kernel scaffold; baseline (speedup 1.0000x reference)
import jax
import jax.numpy as jnp
from jax.experimental import pallas as pl


def kernel(x, edge_index, edge_attr, W, b):
    raise NotImplementedError("write your pallas kernel here")


# speedup vs baseline: 4.1034x; 4.1034x over previous
import functools

import jax
import jax.numpy as jnp
from jax import lax
from jax.experimental import pallas as pl
from jax.experimental.pallas import tpu as pltpu
from jax.experimental.pallas import tpu_sc as plsc

N_NODES = 10000
D_FEAT = 128
D_HALF = 64
D_EDGE = 16
OUT_FEATURES = 128

NPAD = 10240
CHUNK = 128
KG = 160
KA = 80
EPAD = 16 * KG * CHUNK
RPT = NPAD // 16


def _sc_accumulate(xs, rowi, coli, attr):
    mesh = plsc.VectorSubcoreMesh(core_axis_name="c", subcore_axis_name="s")

    @functools.partial(
        pl.kernel,
        out_type=(
            jax.ShapeDtypeStruct((2, NPAD, D_HALF), jnp.float32),
            jax.ShapeDtypeStruct((2, NPAD, D_EDGE), jnp.float32),
            jax.ShapeDtypeStruct((2, NPAD, 16), jnp.float32),
        ),
        mesh=mesh,
        compiler_params=pltpu.CompilerParams(use_tc_tiling_on_sc=False),
        scratch_types=[
            pltpu.VMEM((CHUNK,), jnp.int32),
            pltpu.VMEM((CHUNK,), jnp.int32),
            pltpu.VMEM((CHUNK,), jnp.int32),
            pltpu.VMEM((CHUNK,), jnp.int32),
            pltpu.VMEM((2, CHUNK, D_HALF), jnp.float32),
            pltpu.VMEM((2, CHUNK, D_EDGE), jnp.float32),
            pltpu.VMEM((CHUNK, 16), jnp.float32),
            pltpu.VMEM_SHARED((NPAD, D_HALF), jnp.float32),
            pltpu.VMEM_SHARED((NPAD, D_EDGE), jnp.float32),
            pltpu.VMEM_SHARED((NPAD, 16), jnp.float32),
            pltpu.SemaphoreType.DMA,
            pltpu.SemaphoreType.DMA,
            pltpu.SemaphoreType.DMA,
            pltpu.SemaphoreType.DMA,
            pltpu.SemaphoreType.DMA,
            pltpu.SemaphoreType.DMA,
        ],
    )
    def kern(xs_hbm, rowi_hbm, coli_hbm, attr_hbm, g_out, a_out, cnt_out,
             ri0, ri1, ci0, ci1, xb, ab, ones_b, g_sp, a_sp, cnt_sp,
             semx0, semx1, sema0, sema1, semi0, semi1):
        c = lax.axis_index("c")
        s = lax.axis_index("s")

        @pl.loop(0, CHUNK)
        def _(r):
            ones_b[pl.ds(r, 1), pl.ds(0, 16)] = jnp.ones((1, 16), jnp.float32)
            ab[0, pl.ds(r, 1), pl.ds(0, 16)] = jnp.zeros((1, 16), jnp.float32)

            @pl.loop(0, D_HALF, step=16)
            def _(cc):
                xb[0, pl.ds(r, 1), pl.ds(cc, 16)] = jnp.zeros((1, 16), jnp.float32)

        for k in range(RPT // CHUNK):
            r0 = s * RPT + k * CHUNK
            pltpu.sync_copy(xb.at[0], g_sp.at[pl.ds(r0, CHUNK)])
            pltpu.sync_copy(ab.at[0], a_sp.at[pl.ds(r0, CHUNK)])
            pltpu.sync_copy(ab.at[0], cnt_sp.at[pl.ds(r0, CHUNK)])
        plsc.subcore_barrier()

        ribuf = (ri0, ri1)
        cibuf = (ci0, ci1)
        semx = (semx0, semx1)
        sema = (sema0, sema1)
        semi = (semi0, semi1)

        def fire_idx(j, b):
            pltpu.make_async_copy(rowi_hbm.at[c, s, j], ribuf[b], semi[b]).start()
            pltpu.make_async_copy(coli_hbm.at[s, j], cibuf[b], semi[b]).start()

            @pl.when(j // KA == c)
            def _():
                pltpu.make_async_copy(
                    attr_hbm.at[c, s, j - c * KA], ab.at[b], sema[b]).start()

        def wait_idx(j, b):
            pltpu.make_async_copy(rowi_hbm.at[c, s, j], ribuf[b], semi[b]).wait()
            pltpu.make_async_copy(coli_hbm.at[s, j], cibuf[b], semi[b]).wait()

        def fire_gather(j, b):
            pltpu.make_async_copy(xs_hbm.at[ribuf[b]], xb.at[b], semx[b]).start()

        def consume(j, b):
            pltpu.make_async_copy(xs_hbm.at[ribuf[b]], xb.at[b], semx[b]).wait()
            pltpu.sync_copy(xb.at[b], g_sp.at[cibuf[b]], add=True)

            @pl.when(j // KA == c)
            def _():
                pltpu.make_async_copy(
                    attr_hbm.at[c, s, j - c * KA], ab.at[b], sema[b]).wait()
                pltpu.sync_copy(ab.at[b], a_sp.at[cibuf[b]], add=True)
                pltpu.sync_copy(ones_b, cnt_sp.at[cibuf[b]], add=True)

        fire_idx(0, 0)
        fire_idx(1, 1)
        wait_idx(0, 0)
        fire_gather(0, 0)
        wait_idx(1, 1)
        fire_gather(1, 1)

        @pl.loop(0, KG, step=2)
        def _(j):
            consume(j, 0)

            @pl.when(j + 2 < KG)
            def _():
                fire_idx(j + 2, 0)

            consume(j + 1, 1)

            @pl.when(j + 3 < KG)
            def _():
                fire_idx(j + 3, 1)

            @pl.when(j + 2 < KG)
            def _():
                wait_idx(j + 2, 0)
                fire_gather(j + 2, 0)

            @pl.when(j + 3 < KG)
            def _():
                wait_idx(j + 3, 1)
                fire_gather(j + 3, 1)

        plsc.subcore_barrier()

        for k in range(RPT // CHUNK):
            r0 = s * RPT + k * CHUNK
            pltpu.sync_copy(g_sp.at[pl.ds(r0, CHUNK)], xb.at[0])
            pltpu.sync_copy(xb.at[0], g_out.at[c, pl.ds(r0, CHUNK)])
            pltpu.sync_copy(a_sp.at[pl.ds(r0, CHUNK)], ab.at[0])
            pltpu.sync_copy(ab.at[0], a_out.at[c, pl.ds(r0, CHUNK)])
            pltpu.sync_copy(cnt_sp.at[pl.ds(r0, CHUNK)], ab.at[1])
            pltpu.sync_copy(ab.at[1], cnt_out.at[c, pl.ds(r0, CHUNK)])

    return kern(xs, rowi, coli, attr)


def _tc_finish(g, a, cnt, w1at, w1bt, w2t, bb):
    R = 1024
    grid = NPAD // R

    def body(g_ref, a_ref, c_ref, w1a_ref, w1b_ref, w2_ref, b_ref, o_ref):
        am = a_ref[0] + a_ref[1]
        cm = c_ref[0] + c_ref[1]
        cnt1 = cm[:, :1]
        y = jnp.dot(g_ref[0], w1a_ref[...], preferred_element_type=jnp.float32)
        y = y + jnp.dot(g_ref[1], w1b_ref[...], preferred_element_type=jnp.float32)
        y = y + jnp.dot(am, w2_ref[...], preferred_element_type=jnp.float32)
        y = y + cnt1 * b_ref[...]
        o_ref[...] = y / jnp.maximum(cnt1, 1.0)

    return pl.pallas_call(
        body,
        grid=(grid,),
        in_specs=[
            pl.BlockSpec((2, R, D_HALF), lambda i: (0, i, 0)),
            pl.BlockSpec((2, R, D_EDGE), lambda i: (0, i, 0)),
            pl.BlockSpec((2, R, 16), lambda i: (0, i, 0)),
            pl.BlockSpec((D_HALF, OUT_FEATURES), lambda i: (0, 0)),
            pl.BlockSpec((D_HALF, OUT_FEATURES), lambda i: (0, 0)),
            pl.BlockSpec((D_EDGE, OUT_FEATURES), lambda i: (0, 0)),
            pl.BlockSpec((1, OUT_FEATURES), lambda i: (0, 0)),
        ],
        out_specs=pl.BlockSpec((R, OUT_FEATURES), lambda i: (i, 0)),
        out_shape=jax.ShapeDtypeStruct((NPAD, OUT_FEATURES), jnp.float32),
    )(g, a, cnt, w1at, w1bt, w2t, bb)


def kernel(x, edge_index, edge_attr, W, b):
    row = edge_index[0].astype(jnp.int32)
    col = edge_index[1].astype(jnp.int32)
    e = row.shape[0]
    pad = EPAD - e
    row_p = jnp.concatenate([row, jnp.zeros((pad,), jnp.int32)])
    col_p = jnp.concatenate([col, jnp.full((pad,), NPAD - 1, jnp.int32)])
    attr_p = jnp.concatenate(
        [edge_attr, jnp.zeros((pad, D_EDGE), edge_attr.dtype)])

    xs = jnp.concatenate([x[:, :D_HALF], x[:, D_HALF:]], axis=0)
    rowi0 = row_p.reshape(16, KG, CHUNK)
    rowi = jnp.stack([rowi0, rowi0 + N_NODES])
    coli = col_p.reshape(16, KG, CHUNK)
    attr = attr_p.reshape(16, 2, KA, CHUNK, D_EDGE).transpose(1, 0, 2, 3, 4)

    g, a, cnt = _sc_accumulate(xs, rowi, coli, attr)

    w1at = W[:, :D_HALF].T
    w1bt = W[:, D_HALF:D_FEAT].T
    w2t = W[:, D_FEAT:].T
    bb = b.reshape(1, OUT_FEATURES)
    out_full = _tc_finish(g, a, cnt, w1at, w1bt, w2t, bb)
    return out_full[:N_NODES]


# speedup vs baseline: 4.4678x; 1.0888x over previous
import functools

import jax
import jax.numpy as jnp
from jax import lax
from jax.experimental import pallas as pl
from jax.experimental.pallas import tpu as pltpu
from jax.experimental.pallas import tpu_sc as plsc

N_NODES = 10000
D_FEAT = 128
D_HALF = 64
D_EDGE = 16
OUT_FEATURES = 128

NPAD = 10240
CHUNK = 128
KG = 160
KA = 80
EPAD = 16 * KG * CHUNK
RPT = NPAD // 16


def _sc_accumulate(xs, rowi, coli, attr):
    mesh = plsc.VectorSubcoreMesh(core_axis_name="c", subcore_axis_name="s")

    @functools.partial(
        pl.kernel,
        out_type=(
            jax.ShapeDtypeStruct((2, NPAD, D_HALF), jnp.float32),
            jax.ShapeDtypeStruct((2, NPAD, D_EDGE), jnp.float32),
            jax.ShapeDtypeStruct((2, NPAD, 16), jnp.float32),
        ),
        mesh=mesh,
        compiler_params=pltpu.CompilerParams(use_tc_tiling_on_sc=False),
        scratch_types=[
            pltpu.VMEM((4, CHUNK), jnp.int32),
            pltpu.VMEM((4, CHUNK), jnp.int32),
            pltpu.VMEM((4, CHUNK, D_HALF), jnp.float32),
            pltpu.VMEM((4, CHUNK, D_EDGE), jnp.float32),
            pltpu.VMEM((CHUNK, 16), jnp.float32),
            pltpu.VMEM_SHARED((NPAD, D_HALF), jnp.float32),
            pltpu.VMEM_SHARED((NPAD, D_EDGE), jnp.float32),
            pltpu.VMEM_SHARED((NPAD, 16), jnp.float32),
            [pltpu.SemaphoreType.DMA] * 4,
            [pltpu.SemaphoreType.DMA] * 4,
            [pltpu.SemaphoreType.DMA] * 4,
            [pltpu.SemaphoreType.DMA] * 4,
            [pltpu.SemaphoreType.DMA] * 4,
        ],
    )
    def kern(xs_hbm, rowi_hbm, coli_hbm, attr_hbm, g_out, a_out, cnt_out,
             ri, ci, xb, ab, ones_b, g_sp, a_sp, cnt_sp,
             semi, sema, semx, semg, semsa):
        c = lax.axis_index("c")
        s = lax.axis_index("s")

        @pl.loop(0, CHUNK)
        def _(r):
            ones_b[pl.ds(r, 1), pl.ds(0, 16)] = jnp.ones((1, 16), jnp.float32)
            ab[0, pl.ds(r, 1), pl.ds(0, 16)] = jnp.zeros((1, 16), jnp.float32)

            @pl.loop(0, D_HALF, step=16)
            def _(cc):
                xb[0, pl.ds(r, 1), pl.ds(cc, 16)] = jnp.zeros((1, 16), jnp.float32)

        for k in range(RPT // CHUNK):
            r0 = s * RPT + k * CHUNK
            pltpu.sync_copy(xb.at[0], g_sp.at[pl.ds(r0, CHUNK)])
            pltpu.sync_copy(ab.at[0], a_sp.at[pl.ds(r0, CHUNK)])
            pltpu.sync_copy(ab.at[0], cnt_sp.at[pl.ds(r0, CHUNK)])
        plsc.subcore_barrier()

        def fire_idx(j, k):
            pltpu.make_async_copy(rowi_hbm.at[c, s, j], ri.at[k], semi[k]).start()
            pltpu.make_async_copy(coli_hbm.at[s, j], ci.at[k], semi[k]).start()

            @pl.when(j // KA == c)
            def _():
                pltpu.make_async_copy(
                    attr_hbm.at[c, s, j - c * KA], ab.at[k], sema[k]).start()

        def wait_idx(j, k):
            pltpu.make_async_copy(rowi_hbm.at[c, s, j], ri.at[k], semi[k]).wait()
            pltpu.make_async_copy(coli_hbm.at[s, j], ci.at[k], semi[k]).wait()

        def fire_gather(j, k, b):
            pltpu.make_async_copy(xs_hbm.at[ri.at[k]], xb.at[b], semx[b]).start()

        def fire_scatter(j, k, b):
            pltpu.make_async_copy(xs_hbm.at[ri.at[k]], xb.at[b], semx[b]).wait()
            pltpu.async_copy(xb.at[b], g_sp.at[ci.at[k]], semg[b], add=True)

            @pl.when(j // KA == c)
            def _():
                pltpu.make_async_copy(
                    attr_hbm.at[c, s, j - c * KA], ab.at[k], sema[k]).wait()
                pltpu.async_copy(ab.at[k], a_sp.at[ci.at[k]], semsa[k], add=True)
                pltpu.async_copy(ones_b, cnt_sp.at[ci.at[k]], semsa[k], add=True)

        def wait_scatter(j, k, b):
            pltpu.make_async_copy(xb.at[b], g_sp.at[ci.at[k]], semg[b]).wait()

            @pl.when(j // KA == c)
            def _():
                pltpu.make_async_copy(ab.at[k], a_sp.at[ci.at[k]], semsa[k]).wait()
                pltpu.make_async_copy(ones_b, cnt_sp.at[ci.at[k]], semsa[k]).wait()

        for k in range(4):
            fire_idx(k, k)
        wait_idx(0, 0)
        fire_gather(0, 0, 0)
        wait_idx(1, 1)
        fire_gather(1, 1, 1)

        @pl.loop(0, KG, step=4)
        def _(j):
            fire_scatter(j, 0, 0)
            wait_idx(j + 2, 2)
            fire_gather(j + 2, 2, 2)
            fire_scatter(j + 1, 1, 1)
            wait_idx(j + 3, 3)
            fire_gather(j + 3, 3, 3)

            wait_scatter(j, 0, 0)

            @pl.when(j + 4 < KG)
            def _():
                fire_idx(j + 4, 0)

            fire_scatter(j + 2, 2, 2)
            wait_scatter(j + 1, 1, 1)

            @pl.when(j + 5 < KG)
            def _():
                fire_idx(j + 5, 1)

            fire_scatter(j + 3, 3, 3)
            wait_scatter(j + 2, 2, 2)

            @pl.when(j + 6 < KG)
            def _():
                fire_idx(j + 6, 2)

            @pl.when(j + 4 < KG)
            def _():
                wait_idx(j + 4, 0)
                fire_gather(j + 4, 0, 0)

            wait_scatter(j + 3, 3, 3)

            @pl.when(j + 7 < KG)
            def _():
                fire_idx(j + 7, 3)

            @pl.when(j + 5 < KG)
            def _():
                wait_idx(j + 5, 1)
                fire_gather(j + 5, 1, 1)

        plsc.subcore_barrier()

        for k in range(RPT // CHUNK):
            r0 = s * RPT + k * CHUNK
            pltpu.sync_copy(g_sp.at[pl.ds(r0, CHUNK)], xb.at[0])
            pltpu.sync_copy(xb.at[0], g_out.at[c, pl.ds(r0, CHUNK)])
            pltpu.sync_copy(a_sp.at[pl.ds(r0, CHUNK)], ab.at[0])
            pltpu.sync_copy(ab.at[0], a_out.at[c, pl.ds(r0, CHUNK)])
            pltpu.sync_copy(cnt_sp.at[pl.ds(r0, CHUNK)], ab.at[1])
            pltpu.sync_copy(ab.at[1], cnt_out.at[c, pl.ds(r0, CHUNK)])

    return kern(xs, rowi, coli, attr)


def _tc_finish(g, a, cnt, w1at, w1bt, w2t, bb):
    R = 1024
    grid = NPAD // R

    def body(g_ref, a_ref, c_ref, w1a_ref, w1b_ref, w2_ref, b_ref, o_ref):
        am = a_ref[0] + a_ref[1]
        cm = c_ref[0] + c_ref[1]
        cnt1 = cm[:, :1]
        y = jnp.dot(g_ref[0], w1a_ref[...], preferred_element_type=jnp.float32)
        y = y + jnp.dot(g_ref[1], w1b_ref[...], preferred_element_type=jnp.float32)
        y = y + jnp.dot(am, w2_ref[...], preferred_element_type=jnp.float32)
        y = y + cnt1 * b_ref[...]
        o_ref[...] = y / jnp.maximum(cnt1, 1.0)

    return pl.pallas_call(
        body,
        grid=(grid,),
        in_specs=[
            pl.BlockSpec((2, R, D_HALF), lambda i: (0, i, 0)),
            pl.BlockSpec((2, R, D_EDGE), lambda i: (0, i, 0)),
            pl.BlockSpec((2, R, 16), lambda i: (0, i, 0)),
            pl.BlockSpec((D_HALF, OUT_FEATURES), lambda i: (0, 0)),
            pl.BlockSpec((D_HALF, OUT_FEATURES), lambda i: (0, 0)),
            pl.BlockSpec((D_EDGE, OUT_FEATURES), lambda i: (0, 0)),
            pl.BlockSpec((1, OUT_FEATURES), lambda i: (0, 0)),
        ],
        out_specs=pl.BlockSpec((R, OUT_FEATURES), lambda i: (i, 0)),
        out_shape=jax.ShapeDtypeStruct((NPAD, OUT_FEATURES), jnp.float32),
    )(g, a, cnt, w1at, w1bt, w2t, bb)


def kernel(x, edge_index, edge_attr, W, b):
    row = edge_index[0].astype(jnp.int32)
    col = edge_index[1].astype(jnp.int32)
    e = row.shape[0]
    pad = EPAD - e
    row_p = jnp.concatenate([row, jnp.zeros((pad,), jnp.int32)])
    col_p = jnp.concatenate([col, jnp.full((pad,), NPAD - 1, jnp.int32)])
    attr_p = jnp.concatenate(
        [edge_attr, jnp.zeros((pad, D_EDGE), edge_attr.dtype)])

    xs = jnp.concatenate([x[:, :D_HALF], x[:, D_HALF:]], axis=0)
    rowi0 = row_p.reshape(16, KG, CHUNK)
    rowi = jnp.stack([rowi0, rowi0 + N_NODES])
    coli = col_p.reshape(16, KG, CHUNK)
    attr = attr_p.reshape(16, 2, KA, CHUNK, D_EDGE).transpose(1, 0, 2, 3, 4)

    g, a, cnt = _sc_accumulate(xs, rowi, coli, attr)

    w1at = W[:, :D_HALF].T
    w1bt = W[:, D_HALF:D_FEAT].T
    w2t = W[:, D_FEAT:].T
    bb = b.reshape(1, OUT_FEATURES)
    out_full = _tc_finish(g, a, cnt, w1at, w1bt, w2t, bb)
    return out_full[:N_NODES]
